# baseline (speedup 1.0000x reference)
import functools
import math

import numpy as np
import jax
import jax.numpy as jnp
from jax import lax
from jax.experimental import pallas as pl
from jax.experimental.pallas import tpu as pltpu
from jax.experimental.pallas import tpu_sc as plsc

VOCAB = 100000
D = 768
POS_LEN = 2048
BATCH = 4
SCALE = math.sqrt(float(D))

NC = 2
NS = 16
LANES = 16
NW = NC * NS
B_TOTAL = BATCH * POS_LEN
B_PER_W = B_TOTAL // NW
CHUNK = 64
N_CHUNKS = B_PER_W // CHUNK
VECS_PER_ROW = D // LANES


def _positional_encoding() -> np.ndarray:
    depth = D // 2
    positions = np.arange(POS_LEN)[:, np.newaxis]
    depths = np.arange(depth)[np.newaxis, :] / depth
    angle_rates = 1.0 / 10000.0 ** depths
    angle_rads = positions * angle_rates
    return np.concatenate(
        [np.sin(angle_rads), np.cos(angle_rads)], axis=-1
    ).astype(np.float32)


_POS_NP = _positional_encoding()

_MESH = plsc.VectorSubcoreMesh(
    core_axis_name="c", subcore_axis_name="s", num_cores=NC, num_subcores=NS
)


@functools.partial(
    pl.kernel,
    out_type=jax.ShapeDtypeStruct((B_TOTAL, D), jnp.float32),
    mesh=_MESH,
    scratch_types=[
        pltpu.VMEM((B_PER_W,), jnp.int32),
        pltpu.VMEM((CHUNK, D), jnp.float32),
        pltpu.VMEM((CHUNK, D), jnp.float32),
        pltpu.SemaphoreType.DMA,
    ],
)
def _sc_embed(x_hbm, table_hbm, pos_hbm, out_hbm, idx_v, emb_v, pos_v, sem):
    wid = lax.axis_index("s") * NC + lax.axis_index("c")
    base = wid * B_PER_W
    t0 = (wid % (POS_LEN // B_PER_W)) * B_PER_W

    pltpu.sync_copy(x_hbm.at[pl.ds(base, B_PER_W)], idx_v)

    for c in range(N_CHUNKS):
        rbase = c * CHUNK
        pltpu.async_copy(
            table_hbm.at[idx_v.at[pl.ds(rbase, CHUNK)]], emb_v, sem
        ).wait()
        pltpu.sync_copy(pos_hbm.at[pl.ds(t0 + rbase, CHUNK)], pos_v)

        def row_body(r, _):
            for k in range(VECS_PER_ROW):
                sl = pl.ds(k * LANES, LANES)
                emb_v[r, sl] = emb_v[r, sl] * SCALE + pos_v[r, sl]
            return 0

        lax.fori_loop(0, CHUNK, row_body, 0)
        pltpu.sync_copy(emb_v, out_hbm.at[pl.ds(base + rbase, CHUNK)])


def kernel(x, table):
    pos = jnp.asarray(_POS_NP)
    xf = x.reshape(-1).astype(jnp.int32)
    out = _sc_embed(xf, table, pos)
    return out.reshape(BATCH, POS_LEN, D)
